# baseline (speedup 1.0000x reference)
import functools

import jax
import jax.numpy as jnp
import numpy as np
from jax import lax
from jax.experimental import pallas as pl
from jax.experimental.pallas import tpu as pltpu
from jax.experimental.pallas import tpu_sc as plsc

_B, _N, _D = 16, 100000, 6
_K = 8192
_NC, _NS = 2, 16
_NW = _NC * _NS
_RPW = _B * _K // _NW
_CH = 128
_NCH = _RPW // _CH

def _perm_idx():
    perm = jax.random.permutation(jax.random.key(42), _N)
    return perm[:_K].astype(jnp.int32)


def _abs_idx(idx):
    idx_abs = idx[None, :] + (jnp.arange(_B, dtype=jnp.int32) * _N)[:, None]
    return idx_abs.reshape(_NW, _NCH, _CH)


try:
    with jax.default_device(jax.local_devices(backend="cpu")[0]):
        _IDX_ABS = np.asarray(jax.device_get(_abs_idx(_perm_idx())))
except Exception:
    _IDX_ABS = None


def _sc_gather(table, idx):
    mesh = plsc.VectorSubcoreMesh(core_axis_name="c", subcore_axis_name="s")

    @functools.partial(
        pl.kernel,
        out_type=jax.ShapeDtypeStruct((_B * _K, _D), jnp.float32),
        mesh=mesh,
        scratch_types=[
            pltpu.VMEM((_NCH, _CH), jnp.int32),
            pltpu.VMEM((_RPW, _D), jnp.float32),
            pltpu.SemaphoreType.DMA,
        ],
        compiler_params=pltpu.CompilerParams(use_tc_tiling_on_sc=False),
    )
    def run(table_hbm, idx_hbm, out_hbm, idx_v, rows_v, sem):
        wid = lax.axis_index("s") * _NC + lax.axis_index("c")
        pltpu.sync_copy(idx_hbm.at[wid], idx_v)
        for j in range(_NCH):
            pltpu.async_copy(
                table_hbm.at[idx_v.at[j]],
                rows_v.at[pl.ds(j * _CH, _CH)],
                sem,
            ).wait()
        pltpu.sync_copy(rows_v, out_hbm.at[pl.ds(wid * _RPW, _RPW)])

    return run(table, idx)


def kernel(points):
    table = points.reshape(_B * _N, _D)
    idx = jnp.asarray(_IDX_ABS) if _IDX_ABS is not None else _abs_idx(_perm_idx())
    out = _sc_gather(table, idx)
    return out.reshape(_B, _K, _D)


# speedup vs baseline: 1.0126x; 1.0126x over previous
import functools

import jax
import jax.numpy as jnp
import numpy as np
from jax import lax
from jax.experimental import pallas as pl
from jax.experimental.pallas import tpu as pltpu
from jax.experimental.pallas import tpu_sc as plsc

_B, _N, _D = 16, 100000, 6
_K = 8192
_NC, _NS = 2, 16
_NW = _NC * _NS
_RPW = _B * _K // _NW
_CH = 128
_NCH = _RPW // _CH
_NBUF = 8

def _perm_idx():
    perm = jax.random.permutation(jax.random.key(42), _N)
    return perm[:_K].astype(jnp.int32)


def _abs_idx(idx):
    idx_abs = idx[None, :] + (jnp.arange(_B, dtype=jnp.int32) * _N)[:, None]
    return idx_abs.reshape(_NW, _NCH, _CH)


try:
    with jax.default_device(jax.local_devices(backend="cpu")[0]):
        _IDX_ABS = np.asarray(jax.device_get(_abs_idx(_perm_idx())))
except Exception:
    _IDX_ABS = None


def _sc_gather(table, idx):
    mesh = plsc.VectorSubcoreMesh(core_axis_name="c", subcore_axis_name="s")

    @functools.partial(
        pl.kernel,
        out_type=jax.ShapeDtypeStruct((_B * _K, _D), jnp.float32),
        mesh=mesh,
        scratch_types=[
            pltpu.VMEM((_NCH, _CH), jnp.int32),
            pltpu.VMEM((_RPW, _D), jnp.float32),
            pltpu.SemaphoreType.DMA,
        ],
        compiler_params=pltpu.CompilerParams(use_tc_tiling_on_sc=False),
    )
    def run(table_hbm, idx_hbm, out_hbm, idx_v, rows_v, sem):
        wid = lax.axis_index("s") * _NC + lax.axis_index("c")
        pltpu.sync_copy(idx_hbm.at[wid], idx_v)
        for g in range(_NCH // _NBUF):
            cps = [
                pltpu.async_copy(
                    table_hbm.at[idx_v.at[g * _NBUF + b]],
                    rows_v.at[pl.ds((g * _NBUF + b) * _CH, _CH)],
                    sem,
                )
                for b in range(_NBUF)
            ]
            for cp in cps:
                cp.wait()
        pltpu.sync_copy(rows_v, out_hbm.at[pl.ds(wid * _RPW, _RPW)])

    return run(table, idx)


def kernel(points):
    table = points.reshape(_B * _N, _D)
    idx = jnp.asarray(_IDX_ABS) if _IDX_ABS is not None else _abs_idx(_perm_idx())
    out = _sc_gather(table, idx)
    return out.reshape(_B, _K, _D)


# speedup vs baseline: 29.6273x; 29.2598x over previous
import functools

import jax
import jax.numpy as jnp
import numpy as np
from jax import lax
from jax.experimental import pallas as pl
from jax.experimental.pallas import tpu as pltpu
from jax.experimental.pallas import tpu_sc as plsc

_B, _N, _D = 16, 100000, 6
_K = 8192
_NC, _NS = 2, 16
_NW = _NC * _NS
_NPLANES = _D * _B
_PPW = _NPLANES // _NW
_L = 16
_NG = _K // _L


def _perm_idx():
    perm = jax.random.permutation(jax.random.key(42), _N)
    return perm[:_K].astype(jnp.int32)


try:
    with jax.default_device(jax.local_devices(backend="cpu")[0]):
        _IDX = np.asarray(jax.device_get(_perm_idx()))
except Exception:
    _IDX = None


def _sc_gather(points_t, idx):
    mesh = plsc.VectorSubcoreMesh(core_axis_name="c", subcore_axis_name="s")

    @functools.partial(
        pl.kernel,
        out_type=jax.ShapeDtypeStruct((_D, _B, _K), jnp.float32),
        mesh=mesh,
        scratch_types=[
            pltpu.VMEM((_N,), jnp.float32),
            pltpu.VMEM((_K,), jnp.float32),
            pltpu.VMEM((_K,), jnp.int32),
            pltpu.SemaphoreType.DMA,
        ],
        compiler_params=pltpu.CompilerParams(needs_layout_passes=False),
    )
    def run(pts_hbm, idx_hbm, out_hbm, plane_v, out_v, idx_v, sem):
        wid = lax.axis_index("s") * _NC + lax.axis_index("c")
        pltpu.sync_copy(idx_hbm, idx_v)
        for i in range(_PPW):
            p = wid * _PPW + i
            c = p // _B
            b = p % _B
            pltpu.sync_copy(pts_hbm.at[c, b], plane_v)

            def body(g, _):
                ids = idx_v[pl.ds(g * _L, _L)]
                out_v[pl.ds(g * _L, _L)] = plsc.load_gather(plane_v, [ids])
                return _

            lax.fori_loop(0, _NG, body, 0)
            pltpu.sync_copy(out_v, out_hbm.at[c, b])

    return run(points_t, idx)


def kernel(points):
    idx = jnp.asarray(_IDX) if _IDX is not None else _perm_idx()
    points_t = jnp.transpose(points, (2, 0, 1))
    out_t = _sc_gather(points_t, idx)
    return jnp.transpose(out_t, (1, 2, 0))
